# baseline (speedup 1.0000x reference)
import functools

import jax
import jax.numpy as jnp
from jax import lax
from jax.experimental import pallas as pl
from jax.experimental.pallas import tpu as pltpu
from jax.experimental.pallas import tpu_sc as plsc

D = 1024
NUM_ROWS = 100000

NC, NS, L = 2, 16, 16
NW = NC * NS
SC_ROWS = 32000
RPW = SC_ROWS // NW
SC_CHUNK = 40
SC_NCHUNKS = RPW // SC_CHUNK

TC_ROWS = NUM_ROWS - SC_ROWS
TC_BLOCK = 2000
TC_FIRST_BLOCK = SC_ROWS // TC_BLOCK


def _sc_body(mem_hbm, q_hbm, cand_out, row_out, q_v, chunk_v, bestrow_v, cand_v):
    c = lax.axis_index("c")
    s = lax.axis_index("s")
    wid = c * NS + s
    base = wid * RPW
    pltpu.sync_copy(q_hbm, q_v)

    def chunk_body(ci, carry):
        row0 = base + ci * SC_CHUNK
        pltpu.sync_copy(mem_hbm.at[pl.ds(row0, SC_CHUNK)], chunk_v)

        def row_body(r, carry2):
            bd, bs, bi = carry2
            accd = jnp.zeros((L,), jnp.float32)
            accs = jnp.zeros((L,), jnp.float32)
            for j in range(D // L):
                v = chunk_v[r, pl.ds(j * L, L)]
                qv = q_v[pl.ds(j * L, L)]
                accd = accd + v * qv
                accs = accs + v * v
            d = jnp.sum(accd)
            sq = jnp.sum(accs)
            cge = d >= 0.0
            bge = bd >= 0.0
            ck = d * d * bs
            bk = bd * bd * sq
            better = lax.select(cge != bge, cge,
                                lax.select(cge, ck > bk, ck < bk))

            @pl.when(better)
            def _():
                for j in range(D // L):
                    bestrow_v[pl.ds(j * L, L)] = chunk_v[r, pl.ds(j * L, L)]

            nbd = lax.select(better, d, bd)
            nbs = lax.select(better, sq, bs)
            nbi = lax.select(better, row0 + r, bi)
            return nbd, nbs, nbi

        return lax.fori_loop(0, SC_CHUNK, row_body, carry)

    init = (-jnp.inf, jnp.float32(1.0), jnp.int32(0))
    bd, bs, bi = lax.fori_loop(0, SC_NCHUNKS, chunk_body, init)

    lanes = lax.iota(jnp.int32, L)
    cvec = jnp.where(lanes == 0, bd,
                     jnp.where(lanes == 1, bs,
                               jnp.where(lanes == 2, bi.astype(jnp.float32),
                                         0.0)))
    cand_v[...] = cvec
    pltpu.sync_copy(cand_v, cand_out.at[wid])
    pltpu.sync_copy(bestrow_v, row_out.at[wid])


def _sc_scan(query, memory):
    mesh = plsc.VectorSubcoreMesh(core_axis_name="c", subcore_axis_name="s",
                                  num_cores=NC, num_subcores=NS)
    f = pl.kernel(
        _sc_body,
        out_type=(
            jax.ShapeDtypeStruct((NW, L), jnp.float32),
            jax.ShapeDtypeStruct((NW, D), jnp.float32),
        ),
        mesh=mesh,
        scratch_types=[
            pltpu.VMEM((D,), jnp.float32),
            pltpu.VMEM((SC_CHUNK, D), jnp.float32),
            pltpu.VMEM((D,), jnp.float32),
            pltpu.VMEM((L,), jnp.float32),
        ],
        compiler_params=pltpu.CompilerParams(needs_layout_passes=False),
    )
    return f(memory, query)


def _tc_body(q_ref, mem_ref, out_ref, sim_ref, best_ref):
    i = pl.program_id(0)

    @pl.when(i == 0)
    def _():
        best_ref[0] = -jnp.inf

    q = q_ref[...]
    block = mem_ref[...]
    dn = (((1,), (1,)), ((), ()))
    dots = lax.dot_general(block, q, dn,
                           preferred_element_type=jnp.float32)
    ones = jnp.ones(q.shape, jnp.float32)
    sumsq = lax.dot_general(block * block, ones, dn,
                            preferred_element_type=jnp.float32)
    q_norm = jnp.sqrt(jnp.sum(q * q))
    denom = jnp.maximum(q_norm * jnp.sqrt(sumsq), 1e-8)
    sims = dots / denom
    local_max = jnp.max(sims)

    @pl.when(local_max > best_ref[0])
    def _():
        best_ref[0] = local_max
        r = sims.shape[0]
        iota = lax.broadcasted_iota(jnp.int32, (r, 1), 0)
        idx = jnp.min(jnp.where(sims == local_max, iota, r))
        out_ref[...] = mem_ref[pl.ds(idx, 1), :]

    @pl.when(i == (TC_ROWS // TC_BLOCK) - 1)
    def _():
        sim_ref[...] = jnp.full((1, 1), best_ref[0], jnp.float32)


def _tc_scan(q2, memory):
    grid = TC_ROWS // TC_BLOCK
    return pl.pallas_call(
        _tc_body,
        grid=(grid,),
        in_specs=[
            pl.BlockSpec((1, D), lambda i: (0, 0)),
            pl.BlockSpec((TC_BLOCK, D), lambda i: (TC_FIRST_BLOCK + i, 0)),
        ],
        out_specs=[
            pl.BlockSpec((1, D), lambda i: (0, 0)),
            pl.BlockSpec((1, 1), lambda i: (0, 0)),
        ],
        out_shape=[
            jax.ShapeDtypeStruct((1, D), jnp.float32),
            jax.ShapeDtypeStruct((1, 1), jnp.float32),
        ],
        scratch_shapes=[pltpu.SMEM((1,), jnp.float32)],
    )(q2, memory)


def _merge_body(q_ref, cand_ref, scrows_ref, tcsim_ref, tcrow_ref, out_ref):
    q = q_ref[...]
    q_norm = jnp.sqrt(jnp.sum(q * q))
    d = cand_ref[:, 0:1]
    sq = cand_ref[:, 1:2]
    sims = d / jnp.maximum(q_norm * jnp.sqrt(sq), 1e-8)
    best_sc = jnp.max(sims)
    iota = lax.broadcasted_iota(jnp.int32, (NW, 1), 0)
    w = jnp.min(jnp.where(sims == best_sc, iota, NW))
    sc_row = scrows_ref[pl.ds(w, 1), :]
    tc_better = tcsim_ref[0, 0] > best_sc
    out_ref[...] = jnp.where(tc_better, tcrow_ref[...], sc_row)


def _merge(q2, cand, sc_rows, tc_sim, tc_row):
    return pl.pallas_call(
        _merge_body,
        out_shape=jax.ShapeDtypeStruct((1, D), jnp.float32),
    )(q2, cand, sc_rows, tc_sim, tc_row)


def kernel(query, memory):
    q2 = query.reshape(1, D)
    cand, sc_rows = _sc_scan(query, memory)
    tc_row, tc_sim = _tc_scan(q2, memory)
    out = _merge(q2, cand, sc_rows, tc_sim, tc_row)
    return out.reshape(D)


# speedup vs baseline: 1.3801x; 1.3801x over previous
import functools

import jax
import jax.numpy as jnp
from jax import lax
from jax.experimental import pallas as pl
from jax.experimental.pallas import tpu as pltpu
from jax.experimental.pallas import tpu_sc as plsc

D = 1024
NUM_ROWS = 100000

NC, NS, L = 2, 16, 16
NW = NC * NS
SC_ROWS = 32000
RPW = SC_ROWS // NW
SC_CHUNK = 40
SC_NCHUNKS = RPW // SC_CHUNK
SC_GROUP = 4

TC_ROWS = NUM_ROWS - SC_ROWS
TC_BLOCK = 2000
TC_FIRST_BLOCK = SC_ROWS // TC_BLOCK


def _sc_compute_chunk(chunk_ref, q_v, ci, carry):
    def group_body(g4, carry2):
        bd, bs, bci, bri = carry2
        r = g4 * SC_GROUP
        zero = jnp.zeros((L,), jnp.float32)
        accs_init = tuple([zero] * (2 * SC_GROUP))

        def jblock(jo, accs):
            accs = list(accs)
            for jj in range(8):
                j = jo * 8 + jj
                qv = q_v[pl.ds(j * L, L)]
                for k in range(SC_GROUP):
                    v = chunk_ref[r + k, pl.ds(j * L, L)]
                    accs[2 * k] = accs[2 * k] + v * qv
                    accs[2 * k + 1] = accs[2 * k + 1] + v * v
            return tuple(accs)

        accs = lax.fori_loop(0, (D // L) // 8, jblock, accs_init)
        for k in range(SC_GROUP):
            d = jnp.sum(accs[2 * k])
            sq = jnp.sum(accs[2 * k + 1])
            cge = d >= 0.0
            bge = bd >= 0.0
            ck = d * d * bs
            bk = bd * bd * sq
            better = lax.select(cge != bge, cge,
                                lax.select(cge, ck > bk, ck < bk))
            bd = lax.select(better, d, bd)
            bs = lax.select(better, sq, bs)
            bci = lax.select(better, ci, bci)
            bri = lax.select(better, r + k, bri)
        return bd, bs, bci, bri

    return lax.fori_loop(0, SC_CHUNK // SC_GROUP, group_body, carry)


def _sc_body(mem_hbm, q_hbm, cand_out, row_out, q_v, chunk0_v, chunk1_v,
             cand_v, sem0, sem1):
    c = lax.axis_index("c")
    s = lax.axis_index("s")
    wid = c * NS + s
    base = wid * RPW
    pltpu.sync_copy(q_hbm, q_v)

    def chunk_src(ci):
        return mem_hbm.at[pl.ds(base + ci * SC_CHUNK, SC_CHUNK)]

    def wait(buf, sem):
        pltpu.make_async_copy(chunk_src(0), buf, sem).wait()

    pltpu.async_copy(chunk_src(0), chunk0_v, sem0)

    def pair_body(g, carry):
        ci = g * 2
        pltpu.async_copy(chunk_src(ci + 1), chunk1_v, sem1)
        wait(chunk0_v, sem0)
        carry = _sc_compute_chunk(chunk0_v, q_v, ci, carry)
        pltpu.async_copy(chunk_src(ci + 2), chunk0_v, sem0)
        wait(chunk1_v, sem1)
        return _sc_compute_chunk(chunk1_v, q_v, ci + 1, carry)

    init = (-jnp.inf, jnp.float32(1.0), jnp.int32(0), jnp.int32(0))
    carry = lax.fori_loop(0, (SC_NCHUNKS - 1) // 2, pair_body, init)
    wait(chunk0_v, sem0)
    bd, bs, bci, bri = _sc_compute_chunk(chunk0_v, q_v, SC_NCHUNKS - 1, carry)

    lanes = lax.iota(jnp.int32, L)
    cvec = jnp.where(lanes == 0, bd,
                     jnp.where(lanes == 1, bs, 0.0))
    cand_v[...] = cvec
    pltpu.sync_copy(cand_v, cand_out.at[wid])
    pltpu.sync_copy(chunk_src(bci), chunk1_v)
    pltpu.sync_copy(chunk1_v.at[bri], row_out.at[wid])


def _sc_scan(query, memory):
    mesh = plsc.VectorSubcoreMesh(core_axis_name="c", subcore_axis_name="s",
                                  num_cores=NC, num_subcores=NS)
    f = pl.kernel(
        _sc_body,
        out_type=(
            jax.ShapeDtypeStruct((NW, L), jnp.float32),
            jax.ShapeDtypeStruct((NW, D), jnp.float32),
        ),
        mesh=mesh,
        scratch_types=[
            pltpu.VMEM((D,), jnp.float32),
            pltpu.VMEM((SC_CHUNK, D), jnp.float32),
            pltpu.VMEM((SC_CHUNK, D), jnp.float32),
            pltpu.VMEM((L,), jnp.float32),
            pltpu.SemaphoreType.DMA,
            pltpu.SemaphoreType.DMA,
        ],
        compiler_params=pltpu.CompilerParams(needs_layout_passes=False),
    )
    return f(memory, query)


def _tc_body(q_ref, mem_ref, out_ref, sim_ref, best_ref):
    i = pl.program_id(0)

    @pl.when(i == 0)
    def _():
        best_ref[0] = -jnp.inf

    q = q_ref[...]
    block = mem_ref[...]
    dn = (((1,), (1,)), ((), ()))
    dots = lax.dot_general(block, q, dn,
                           preferred_element_type=jnp.float32)
    ones = jnp.ones(q.shape, jnp.float32)
    sumsq = lax.dot_general(block * block, ones, dn,
                            preferred_element_type=jnp.float32)
    q_norm = jnp.sqrt(jnp.sum(q * q))
    denom = jnp.maximum(q_norm * jnp.sqrt(sumsq), 1e-8)
    sims = dots / denom
    local_max = jnp.max(sims)

    @pl.when(local_max > best_ref[0])
    def _():
        best_ref[0] = local_max
        r = sims.shape[0]
        iota = lax.broadcasted_iota(jnp.int32, (r, 1), 0)
        idx = jnp.min(jnp.where(sims == local_max, iota, r))
        out_ref[...] = mem_ref[pl.ds(idx, 1), :]

    @pl.when(i == (TC_ROWS // TC_BLOCK) - 1)
    def _():
        sim_ref[...] = jnp.full((1, 1), best_ref[0], jnp.float32)


def _tc_scan(q2, memory):
    grid = TC_ROWS // TC_BLOCK
    return pl.pallas_call(
        _tc_body,
        grid=(grid,),
        in_specs=[
            pl.BlockSpec((1, D), lambda i: (0, 0)),
            pl.BlockSpec((TC_BLOCK, D), lambda i: (TC_FIRST_BLOCK + i, 0)),
        ],
        out_specs=[
            pl.BlockSpec((1, D), lambda i: (0, 0)),
            pl.BlockSpec((1, 1), lambda i: (0, 0)),
        ],
        out_shape=[
            jax.ShapeDtypeStruct((1, D), jnp.float32),
            jax.ShapeDtypeStruct((1, 1), jnp.float32),
        ],
        scratch_shapes=[pltpu.SMEM((1,), jnp.float32)],
    )(q2, memory)


def _merge_body(q_ref, cand_ref, scrows_ref, tcsim_ref, tcrow_ref, out_ref):
    q = q_ref[...]
    q_norm = jnp.sqrt(jnp.sum(q * q))
    d = cand_ref[:, 0:1]
    sq = cand_ref[:, 1:2]
    sims = d / jnp.maximum(q_norm * jnp.sqrt(sq), 1e-8)
    best_sc = jnp.max(sims)
    iota = lax.broadcasted_iota(jnp.int32, (NW, 1), 0)
    w = jnp.min(jnp.where(sims == best_sc, iota, NW))
    sc_row = scrows_ref[pl.ds(w, 1), :]
    tc_better = tcsim_ref[0, 0] > best_sc
    out_ref[...] = jnp.where(tc_better, tcrow_ref[...], sc_row)


def _merge(q2, cand, sc_rows, tc_sim, tc_row):
    return pl.pallas_call(
        _merge_body,
        out_shape=jax.ShapeDtypeStruct((1, D), jnp.float32),
    )(q2, cand, sc_rows, tc_sim, tc_row)


def kernel(query, memory):
    q2 = query.reshape(1, D)
    cand, sc_rows = _sc_scan(query, memory)
    tc_row, tc_sim = _tc_scan(q2, memory)
    out = _merge(q2, cand, sc_rows, tc_sim, tc_row)
    return out.reshape(D)


# speedup vs baseline: 1.4171x; 1.0268x over previous
import functools

import jax
import jax.numpy as jnp
from jax import lax
from jax.experimental import pallas as pl
from jax.experimental.pallas import tpu as pltpu
from jax.experimental.pallas import tpu_sc as plsc

D = 1024
NUM_ROWS = 100000

NC, NS, L = 2, 16, 16
NW = NC * NS
SC_ROWS = 40960
RPW = SC_ROWS // NW
SC_CHUNK = 40
SC_NCHUNKS = RPW // SC_CHUNK
SC_GROUP = 4

TC_BLOCK = 2000
TC_START = 40000
TC_ROWS = NUM_ROWS - TC_START
TC_FIRST_BLOCK = TC_START // TC_BLOCK


def _sc_compute_chunk(chunk_ref, q_v, ci, carry):
    def group_body(g4, carry2):
        bd, bs, bci, bri = carry2
        r = g4 * SC_GROUP
        zero = jnp.zeros((L,), jnp.float32)
        accs_init = tuple([zero] * (2 * SC_GROUP))

        def jblock(jo, accs):
            accs = list(accs)
            for jj in range(8):
                j = jo * 8 + jj
                qv = q_v[pl.ds(j * L, L)]
                for k in range(SC_GROUP):
                    v = chunk_ref[r + k, pl.ds(j * L, L)]
                    accs[2 * k] = accs[2 * k] + v * qv
                    accs[2 * k + 1] = accs[2 * k + 1] + v * v
            return tuple(accs)

        accs = lax.fori_loop(0, (D // L) // 8, jblock, accs_init)
        for k in range(SC_GROUP):
            d = jnp.sum(accs[2 * k])
            sq = jnp.sum(accs[2 * k + 1])
            cge = d >= 0.0
            bge = bd >= 0.0
            ck = d * d * bs
            bk = bd * bd * sq
            better = lax.select(cge != bge, cge,
                                lax.select(cge, ck > bk, ck < bk))
            bd = lax.select(better, d, bd)
            bs = lax.select(better, sq, bs)
            bci = lax.select(better, ci, bci)
            bri = lax.select(better, r + k, bri)
        return bd, bs, bci, bri

    return lax.fori_loop(0, SC_CHUNK // SC_GROUP, group_body, carry)


def _sc_body(mem_hbm, q_hbm, cand_out, row_out, q_v, chunk0_v, chunk1_v,
             cand_v, sem0, sem1):
    c = lax.axis_index("c")
    s = lax.axis_index("s")
    wid = c * NS + s
    base = wid * RPW
    pltpu.sync_copy(q_hbm, q_v)

    def chunk_src(ci):
        return mem_hbm.at[pl.ds(base + ci * SC_CHUNK, SC_CHUNK)]

    def wait(buf, sem):
        pltpu.make_async_copy(chunk_src(0), buf, sem).wait()

    pltpu.async_copy(chunk_src(0), chunk0_v, sem0)

    def pair_body(g, carry):
        ci = g * 2
        pltpu.async_copy(chunk_src(ci + 1), chunk1_v, sem1)
        wait(chunk0_v, sem0)
        carry = _sc_compute_chunk(chunk0_v, q_v, ci, carry)

        @pl.when(ci + 2 < SC_NCHUNKS)
        def _():
            pltpu.async_copy(chunk_src(ci + 2), chunk0_v, sem0)

        wait(chunk1_v, sem1)
        return _sc_compute_chunk(chunk1_v, q_v, ci + 1, carry)

    init = (-jnp.inf, jnp.float32(1.0), jnp.int32(0), jnp.int32(0))
    bd, bs, bci, bri = lax.fori_loop(0, SC_NCHUNKS // 2, pair_body, init)

    lanes = lax.iota(jnp.int32, L)
    cvec = jnp.where(lanes == 0, bd,
                     jnp.where(lanes == 1, bs, 0.0))
    cand_v[...] = cvec
    pltpu.sync_copy(cand_v, cand_out.at[wid])
    pltpu.sync_copy(chunk_src(bci), chunk1_v)
    pltpu.sync_copy(chunk1_v.at[bri], row_out.at[wid])


def _sc_scan(query, memory):
    mesh = plsc.VectorSubcoreMesh(core_axis_name="c", subcore_axis_name="s",
                                  num_cores=NC, num_subcores=NS)
    f = pl.kernel(
        _sc_body,
        out_type=(
            jax.ShapeDtypeStruct((NW, L), jnp.float32),
            jax.ShapeDtypeStruct((NW, D), jnp.float32),
        ),
        mesh=mesh,
        scratch_types=[
            pltpu.VMEM((D,), jnp.float32),
            pltpu.VMEM((SC_CHUNK, D), jnp.float32),
            pltpu.VMEM((SC_CHUNK, D), jnp.float32),
            pltpu.VMEM((L,), jnp.float32),
            pltpu.SemaphoreType.DMA,
            pltpu.SemaphoreType.DMA,
        ],
        compiler_params=pltpu.CompilerParams(needs_layout_passes=False),
    )
    return f(memory, query)


def _tc_body(q_ref, mem_ref, out_ref, sim_ref, best_ref):
    i = pl.program_id(0)

    @pl.when(i == 0)
    def _():
        best_ref[0] = -jnp.inf

    q = q_ref[...]
    block = mem_ref[...]
    dn = (((1,), (1,)), ((), ()))
    dots = lax.dot_general(block, q, dn,
                           preferred_element_type=jnp.float32)
    ones = jnp.ones(q.shape, jnp.float32)
    sumsq = lax.dot_general(block * block, ones, dn,
                            preferred_element_type=jnp.float32)
    q_norm = jnp.sqrt(jnp.sum(q * q))
    denom = jnp.maximum(q_norm * jnp.sqrt(sumsq), 1e-8)
    sims = dots / denom
    local_max = jnp.max(sims)

    @pl.when(local_max > best_ref[0])
    def _():
        best_ref[0] = local_max
        r = sims.shape[0]
        iota = lax.broadcasted_iota(jnp.int32, (r, 1), 0)
        idx = jnp.min(jnp.where(sims == local_max, iota, r))
        out_ref[...] = mem_ref[pl.ds(idx, 1), :]

    @pl.when(i == (TC_ROWS // TC_BLOCK) - 1)
    def _():
        sim_ref[...] = jnp.full((1, 1), best_ref[0], jnp.float32)


def _tc_scan(q2, memory):
    grid = TC_ROWS // TC_BLOCK
    return pl.pallas_call(
        _tc_body,
        grid=(grid,),
        in_specs=[
            pl.BlockSpec((1, D), lambda i: (0, 0)),
            pl.BlockSpec((TC_BLOCK, D), lambda i: (TC_FIRST_BLOCK + i, 0)),
        ],
        out_specs=[
            pl.BlockSpec((1, D), lambda i: (0, 0)),
            pl.BlockSpec((1, 1), lambda i: (0, 0)),
        ],
        out_shape=[
            jax.ShapeDtypeStruct((1, D), jnp.float32),
            jax.ShapeDtypeStruct((1, 1), jnp.float32),
        ],
        scratch_shapes=[pltpu.SMEM((1,), jnp.float32)],
    )(q2, memory)


def _merge_body(q_ref, cand_ref, scrows_ref, tcsim_ref, tcrow_ref, out_ref):
    q = q_ref[...]
    q_norm = jnp.sqrt(jnp.sum(q * q))
    d = cand_ref[:, 0:1]
    sq = cand_ref[:, 1:2]
    sims = d / jnp.maximum(q_norm * jnp.sqrt(sq), 1e-8)
    best_sc = jnp.max(sims)
    iota = lax.broadcasted_iota(jnp.int32, (NW, 1), 0)
    w = jnp.min(jnp.where(sims == best_sc, iota, NW))
    sc_row = scrows_ref[pl.ds(w, 1), :]
    tc_better = tcsim_ref[0, 0] > best_sc
    out_ref[...] = jnp.where(tc_better, tcrow_ref[...], sc_row)


def _merge(q2, cand, sc_rows, tc_sim, tc_row):
    return pl.pallas_call(
        _merge_body,
        out_shape=jax.ShapeDtypeStruct((1, D), jnp.float32),
    )(q2, cand, sc_rows, tc_sim, tc_row)


def kernel(query, memory):
    q2 = query.reshape(1, D)
    cand, sc_rows = _sc_scan(query, memory)
    tc_row, tc_sim = _tc_scan(q2, memory)
    out = _merge(q2, cand, sc_rows, tc_sim, tc_row)
    return out.reshape(D)
